# baseline (speedup 1.0000x reference)
import functools

import jax
import jax.numpy as jnp
import numpy as np
from jax import lax
from jax.experimental import pallas as pl
from jax.experimental.pallas import tpu as pltpu
from jax.experimental.pallas import tpu_sc as plsc

N = 10000
E = 320000
D_IN = 128
D_HID = 128
D_OUT = 40
EPS = 1e-5
CONF_W = 48
C_CHUNK = 40
N_SUBCORES = 32
E_PER_W = E // N_SUBCORES
N_CHUNKS = E_PER_W // C_CHUNK
CPB = 10
ZB_ROWS = 40
N_BLOCKS = N // ZB_ROWS


def _dense_stage(hb, lw, lb, w, b, dp):
    logits = jnp.dot(hb, lw, preferred_element_type=jnp.float32) + lb
    m = jnp.max(logits, axis=1, keepdims=True)
    e = jnp.exp(logits - m)
    conf = e / jnp.sum(e, axis=1, keepdims=True)
    r = hb.shape[0]
    h2 = jnp.dot(hb, w, preferred_element_type=jnp.float32) + b
    do = h2.shape[1]
    pad = dp - do - 1
    conf48 = jnp.concatenate(
        [conf, jnp.zeros((r, CONF_W - conf.shape[1]), jnp.float32)], axis=1)
    comb = jnp.concatenate(
        [h2, jnp.ones((r, 1), jnp.float32),
         jnp.zeros((r, pad), jnp.float32), conf48], axis=1)
    return comb, conf48


def _tc_prep_body(h_ref, lw_ref, lb_ref, w_ref, b_ref, comb_ref, conf_ref,
                  *, dp):
    comb, conf48 = _dense_stage(h_ref[...], lw_ref[...], lb_ref[...],
                                w_ref[...], b_ref[...], dp)
    comb_ref[...] = comb
    conf_ref[...] = conf48


def _tc_prep(h, lw, lb, w, b, dp, grid_r=1000):
    g = h.shape[0] // grid_r
    return pl.pallas_call(
        functools.partial(_tc_prep_body, dp=dp),
        grid=(g,),
        in_specs=[
            pl.BlockSpec((grid_r, h.shape[1]), lambda i: (i, 0)),
            pl.BlockSpec(lw.shape, lambda i: (0, 0)),
            pl.BlockSpec(lb.shape, lambda i: (0, 0)),
            pl.BlockSpec(w.shape, lambda i: (0, 0)),
            pl.BlockSpec(b.shape, lambda i: (0, 0)),
        ],
        out_specs=[
            pl.BlockSpec((grid_r, dp + CONF_W), lambda i: (i, 0)),
            pl.BlockSpec((grid_r, CONF_W), lambda i: (i, 0)),
        ],
        out_shape=[
            jax.ShapeDtypeStruct((h.shape[0], dp + CONF_W), jnp.float32),
            jax.ShapeDtypeStruct((h.shape[0], CONF_W), jnp.float32),
        ],
    )(h, lw, lb, w, b)


def _combine(part_ref, comb_ref, do):
    tot = part_ref[0] + part_ref[1]
    h2 = comb_ref[...][:, 0:do]
    agg = tot[:, 0:do]
    deg = tot[:, do:do + 1]
    return (h2 + agg) / (deg + 1.0)


def _tc_mid_body(part_ref, comb_ref, g_ref, be_ref, lw_ref, lb_ref, w_ref,
                 b_ref, combn_ref, confn_ref, *, dp_next):
    hn = _combine(part_ref, comb_ref, D_HID)
    hn = hn * (g_ref[...] * np.float32(1.0 / np.sqrt(1.0 + EPS))) + be_ref[...]
    hn = jnp.maximum(hn, 0.0)
    comb, conf48 = _dense_stage(hn, lw_ref[...], lb_ref[...], w_ref[...],
                                b_ref[...], dp_next)
    combn_ref[...] = comb
    confn_ref[...] = conf48


def _tc_mid(part, comb, g2d, be2d, lw, lb, w, b, dp_next, grid_r=1000):
    gr = N // grid_r
    dp = part.shape[2]
    return pl.pallas_call(
        functools.partial(_tc_mid_body, dp_next=dp_next),
        grid=(gr,),
        in_specs=[
            pl.BlockSpec((2, grid_r, dp), lambda i: (0, i, 0)),
            pl.BlockSpec((grid_r, comb.shape[1]), lambda i: (i, 0)),
            pl.BlockSpec(g2d.shape, lambda i: (0, 0)),
            pl.BlockSpec(be2d.shape, lambda i: (0, 0)),
            pl.BlockSpec(lw.shape, lambda i: (0, 0)),
            pl.BlockSpec(lb.shape, lambda i: (0, 0)),
            pl.BlockSpec(w.shape, lambda i: (0, 0)),
            pl.BlockSpec(b.shape, lambda i: (0, 0)),
        ],
        out_specs=[
            pl.BlockSpec((grid_r, dp_next + CONF_W), lambda i: (i, 0)),
            pl.BlockSpec((grid_r, CONF_W), lambda i: (i, 0)),
        ],
        out_shape=[
            jax.ShapeDtypeStruct((N, dp_next + CONF_W), jnp.float32),
            jax.ShapeDtypeStruct((N, CONF_W), jnp.float32),
        ],
    )(part, comb, g2d, be2d, lw, lb, w, b)


def _tc_final_body(part_ref, comb_ref, out_ref):
    out_ref[...] = _combine(part_ref, comb_ref, D_OUT)


def _tc_final(part, comb, grid_r=1000):
    gr = N // grid_r
    dp = part.shape[2]
    return pl.pallas_call(
        _tc_final_body,
        grid=(gr,),
        in_specs=[
            pl.BlockSpec((2, grid_r, dp), lambda i: (0, i, 0)),
            pl.BlockSpec((grid_r, comb.shape[1]), lambda i: (i, 0)),
        ],
        out_specs=pl.BlockSpec((grid_r, D_OUT), lambda i: (i, 0)),
        out_shape=jax.ShapeDtypeStruct((N, D_OUT), jnp.float32),
    )(part, comb)


def _sc_body(comb_h, conf_h, src_h, dst_h, out_h,
             rows0, ro0, cd0, rows1, ro1, cd1, sb, db,
             agg_sh, gsem0, gsem1, ssem0, ssem1, *, dp):
    c = lax.axis_index("c")
    s = lax.axis_index("s")
    wid = c * 16 + s
    nv = dp // 16
    sets = ((rows0, ro0, cd0, gsem0, ssem0),
            (rows1, ro1, cd1, gsem1, ssem1))

    zv = jnp.zeros((16,), jnp.float32)

    @pl.loop(0, ZB_ROWS)
    def _(r):
        for v in range(nv):
            ro0[r, pl.ds(v * 16, 16)] = zv

    @pl.loop(s, N_BLOCKS, step=16)
    def _(k):
        pltpu.sync_copy(ro0, agg_sh.at[pl.ds(k * ZB_ROWS, ZB_ROWS)])

    plsc.subcore_barrier()

    def issue(b, i):
        rows, ro, cd, gsem, _ = sets[b]
        pltpu.async_copy(comb_h.at[sb.at[i]], rows, gsem)
        pltpu.async_copy(conf_h.at[db.at[i]], cd, gsem)

    def wait_gathers(b, i):
        rows, ro, cd, gsem, _ = sets[b]
        pltpu.make_async_copy(comb_h.at[sb.at[i]], rows, gsem).wait()
        pltpu.make_async_copy(conf_h.at[db.at[i]], cd, gsem).wait()

    def scatter_async(b, i):
        rows, ro, cd, _, ssem = sets[b]
        pltpu.async_copy(ro, agg_sh.at[db.at[i]], ssem, add=True)

    def wait_scatter(b, i):
        rows, ro, cd, _, ssem = sets[b]
        pltpu.make_async_copy(ro, agg_sh.at[db.at[i]], ssem).wait()

    dn = lax.GatherDimensionNumbers(
        offset_dims=(), collapsed_slice_dims=(0,), start_index_map=(0,))

    def compute(b):
        rows, ro, cd, _, _ = sets[b]

        @pl.loop(0, C_CHUNK, unroll=2)
        def _(e):
            p = (rows[e, pl.ds(dp, 16)] * cd[e, pl.ds(0, 16)]
                 + rows[e, pl.ds(dp + 16, 16)] * cd[e, pl.ds(16, 16)]
                 + rows[e, pl.ds(dp + 32, 16)] * cd[e, pl.ds(32, 16)])
            for stp in (8, 4, 2, 1):
                perm = (jnp.arange(16, dtype=jnp.int32) ^ stp)[:, None]
                p = p + lax.gather(p, perm, dimension_numbers=dn,
                                   slice_sizes=(1,),
                                   mode=lax.GatherScatterMode.PROMISE_IN_BOUNDS)
            for v in range(nv):
                ro[e, pl.ds(v * 16, 16)] = rows[e, pl.ds(v * 16, 16)] * p

    @pl.loop(0, N_CHUNKS // CPB)
    def _(q):
        row0 = wid * N_CHUNKS + q * CPB
        pltpu.sync_copy(src_h.at[pl.ds(row0, CPB)], sb)
        pltpu.sync_copy(dst_h.at[pl.ds(row0, CPB)], db)
        issue(0, 0)
        for i in range(CPB - 1):
            a = i % 2
            nb = 1 - a
            wait_gathers(a, i)
            if i >= 1:
                wait_scatter(nb, i - 1)
            issue(nb, i + 1)
            compute(a)
            scatter_async(a, i)
        a = (CPB - 1) % 2
        wait_gathers(a, CPB - 1)
        compute(a)
        scatter_async(a, CPB - 1)
        wait_scatter(1 - a, CPB - 2)
        wait_scatter(a, CPB - 1)

    plsc.subcore_barrier()

    @pl.loop(s, N_BLOCKS, step=16)
    def _(k):
        r = k * ZB_ROWS
        pltpu.sync_copy(agg_sh.at[pl.ds(r, ZB_ROWS)], ro0)
        pltpu.sync_copy(ro0, out_h.at[c, pl.ds(r, ZB_ROWS)])


def _sc_agg(comb, conf48, src, dst, dp):
    mesh = plsc.VectorSubcoreMesh(core_axis_name="c", subcore_axis_name="s")
    wc = dp + CONF_W
    kern = functools.partial(
        pl.kernel,
        out_type=jax.ShapeDtypeStruct((2, N, dp), jnp.float32),
        mesh=mesh,
        compiler_params=pltpu.CompilerParams(use_tc_tiling_on_sc=False),
        scratch_types=(
            [pltpu.VMEM((C_CHUNK, wc), jnp.float32),
             pltpu.VMEM((C_CHUNK, dp), jnp.float32),
             pltpu.VMEM((C_CHUNK, CONF_W), jnp.float32)] * 2
            + [pltpu.VMEM((CPB, C_CHUNK), jnp.int32),
               pltpu.VMEM((CPB, C_CHUNK), jnp.int32)]
            + [pltpu.VMEM_SHARED((N, dp), jnp.float32)]
            + [pltpu.SemaphoreType.DMA] * 4
        ),
    )(functools.partial(_sc_body, dp=dp))
    return kern(comb, conf48, src, dst)


def kernel(x, edge_index, W0, b0, W1, b1, W2, b2, L0w, L0b, L1w, L1b, L2w,
           L2b, g0, be0, g1, be1):
    src = edge_index[0].reshape(E // C_CHUNK, C_CHUNK)
    dst = edge_index[1].reshape(E // C_CHUNK, C_CHUNK)
    dp = D_HID + 16
    dp2 = D_OUT + 8

    comb0, conf0 = _tc_prep(x, L0w, L0b.reshape(1, -1), W0,
                            b0.reshape(1, -1), dp)
    part0 = _sc_agg(comb0, conf0, src, dst, dp)
    comb1, conf1 = _tc_mid(part0, comb0, g0.reshape(1, -1), be0.reshape(1, -1),
                           L1w, L1b.reshape(1, -1), W1, b1.reshape(1, -1), dp)
    part1 = _sc_agg(comb1, conf1, src, dst, dp)
    comb2, conf2 = _tc_mid(part1, comb1, g1.reshape(1, -1), be1.reshape(1, -1),
                           L2w, L2b.reshape(1, -1), W2, b2.reshape(1, -1), dp2)
    part2 = _sc_agg(comb2, conf2, src, dst, dp2)
    return _tc_final(part2, comb2)


# speedup vs baseline: 1.4501x; 1.4501x over previous
import functools

import jax
import jax.numpy as jnp
import numpy as np
from jax import lax
from jax.experimental import pallas as pl
from jax.experimental.pallas import tpu as pltpu
from jax.experimental.pallas import tpu_sc as plsc

N = 10000
E = 320000
D_IN = 128
D_HID = 128
D_OUT = 40
EPS = 1e-5
CONF_W = 48
C_CHUNK = 80
N_SUBCORES = 32
E_PER_W = E // N_SUBCORES
N_CHUNKS = E_PER_W // C_CHUNK
CPB = 5
ZB_ROWS = 80
N_BLOCKS = N // ZB_ROWS

WA_HID, WB_HID = 80, 64
WA_OUT, WB_OUT = 32, 16


def _dense_stage(hb, lw, lb, w, b, wa, wb):
    logits = jnp.dot(hb, lw, preferred_element_type=jnp.float32) + lb
    m = jnp.max(logits, axis=1, keepdims=True)
    e = jnp.exp(logits - m)
    conf = e / jnp.sum(e, axis=1, keepdims=True)
    r = hb.shape[0]
    conf48 = jnp.concatenate(
        [conf, jnp.zeros((r, CONF_W - conf.shape[1]), jnp.float32)], axis=1)
    h2 = jnp.dot(hb, w, preferred_element_type=jnp.float32) + b
    do = h2.shape[1]
    h2a = h2[:, 0:wa]
    pad = wa + wb - do - 1
    h2b = jnp.concatenate(
        [h2[:, wa:do], jnp.ones((r, 1), jnp.float32),
         jnp.zeros((r, pad), jnp.float32)], axis=1)
    return conf48, h2a, h2b


def _tc_prep_body(h_ref, lw_ref, lb_ref, w_ref, b_ref, conf_ref, ha_ref,
                  hb_ref, *, wa, wb):
    conf48, h2a, h2b = _dense_stage(h_ref[...], lw_ref[...], lb_ref[...],
                                    w_ref[...], b_ref[...], wa, wb)
    conf_ref[...] = conf48
    ha_ref[...] = h2a
    hb_ref[...] = h2b


def _tc_prep(h, lw, lb, w, b, wa, wb, grid_r=1000):
    g = h.shape[0] // grid_r
    return pl.pallas_call(
        functools.partial(_tc_prep_body, wa=wa, wb=wb),
        grid=(g,),
        in_specs=[
            pl.BlockSpec((grid_r, h.shape[1]), lambda i: (i, 0)),
            pl.BlockSpec(lw.shape, lambda i: (0, 0)),
            pl.BlockSpec(lb.shape, lambda i: (0, 0)),
            pl.BlockSpec(w.shape, lambda i: (0, 0)),
            pl.BlockSpec(b.shape, lambda i: (0, 0)),
        ],
        out_specs=[
            pl.BlockSpec((grid_r, CONF_W), lambda i: (i, 0)),
            pl.BlockSpec((grid_r, wa), lambda i: (i, 0)),
            pl.BlockSpec((grid_r, wb), lambda i: (i, 0)),
        ],
        out_shape=[
            jax.ShapeDtypeStruct((h.shape[0], CONF_W), jnp.float32),
            jax.ShapeDtypeStruct((h.shape[0], wa), jnp.float32),
            jax.ShapeDtypeStruct((h.shape[0], wb), jnp.float32),
        ],
    )(h, lw, lb, w, b)


def _combine(pa_ref, pb_ref, ha_ref, hb_ref, do):
    ta = pa_ref[0] + pa_ref[1]
    tb = pb_ref[0] + pb_ref[1]
    wa = ta.shape[1]
    h2 = jnp.concatenate([ha_ref[...], hb_ref[...][:, 0:do - wa]], axis=1)
    agg = jnp.concatenate([ta, tb[:, 0:do - wa]], axis=1)
    deg = tb[:, do - wa:do - wa + 1]
    return (h2 + agg) / (deg + 1.0)


def _tc_mid_body(pa_ref, pb_ref, ha_ref, hb_ref, g_ref, be_ref, lw_ref,
                 lb_ref, w_ref, b_ref, conf_ref, han_ref, hbn_ref,
                 *, wa_next, wb_next):
    hn = _combine(pa_ref, pb_ref, ha_ref, hb_ref, D_HID)
    hn = hn * (g_ref[...] * np.float32(1.0 / np.sqrt(1.0 + EPS))) + be_ref[...]
    hn = jnp.maximum(hn, 0.0)
    conf48, h2a, h2b = _dense_stage(hn, lw_ref[...], lb_ref[...], w_ref[...],
                                    b_ref[...], wa_next, wb_next)
    conf_ref[...] = conf48
    han_ref[...] = h2a
    hbn_ref[...] = h2b


def _tc_mid(pa, pb, ha, hb, g2d, be2d, lw, lb, w, b, wa_next, wb_next,
            grid_r=1000):
    gr = N // grid_r
    wa = pa.shape[2]
    wb = pb.shape[2]
    return pl.pallas_call(
        functools.partial(_tc_mid_body, wa_next=wa_next, wb_next=wb_next),
        grid=(gr,),
        in_specs=[
            pl.BlockSpec((2, grid_r, wa), lambda i: (0, i, 0)),
            pl.BlockSpec((2, grid_r, wb), lambda i: (0, i, 0)),
            pl.BlockSpec((grid_r, wa), lambda i: (i, 0)),
            pl.BlockSpec((grid_r, wb), lambda i: (i, 0)),
            pl.BlockSpec(g2d.shape, lambda i: (0, 0)),
            pl.BlockSpec(be2d.shape, lambda i: (0, 0)),
            pl.BlockSpec(lw.shape, lambda i: (0, 0)),
            pl.BlockSpec(lb.shape, lambda i: (0, 0)),
            pl.BlockSpec(w.shape, lambda i: (0, 0)),
            pl.BlockSpec(b.shape, lambda i: (0, 0)),
        ],
        out_specs=[
            pl.BlockSpec((grid_r, CONF_W), lambda i: (i, 0)),
            pl.BlockSpec((grid_r, wa_next), lambda i: (i, 0)),
            pl.BlockSpec((grid_r, wb_next), lambda i: (i, 0)),
        ],
        out_shape=[
            jax.ShapeDtypeStruct((N, CONF_W), jnp.float32),
            jax.ShapeDtypeStruct((N, wa_next), jnp.float32),
            jax.ShapeDtypeStruct((N, wb_next), jnp.float32),
        ],
    )(pa, pb, ha, hb, g2d, be2d, lw, lb, w, b)


def _tc_final_body(pa_ref, pb_ref, ha_ref, hb_ref, out_ref):
    out_ref[...] = _combine(pa_ref, pb_ref, ha_ref, hb_ref, D_OUT)


def _tc_final(pa, pb, ha, hb, grid_r=1000):
    gr = N // grid_r
    wa = pa.shape[2]
    wb = pb.shape[2]
    return pl.pallas_call(
        _tc_final_body,
        grid=(gr,),
        in_specs=[
            pl.BlockSpec((2, grid_r, wa), lambda i: (0, i, 0)),
            pl.BlockSpec((2, grid_r, wb), lambda i: (0, i, 0)),
            pl.BlockSpec((grid_r, wa), lambda i: (i, 0)),
            pl.BlockSpec((grid_r, wb), lambda i: (i, 0)),
        ],
        out_specs=pl.BlockSpec((grid_r, D_OUT), lambda i: (i, 0)),
        out_shape=jax.ShapeDtypeStruct((N, D_OUT), jnp.float32),
    )(pa, pb, ha, hb)


def _sc_body(conf_h, ha_h, hb_h, src_h, dst_h, outa_h, outb_h,
             cs0, cd0, fa0, fb0, cs1, cd1, fa1, fb1, sb, db,
             agga_sh, aggb_sh, gsem0, gsem1, ssem0, ssem1, *, wa, wb):
    c = lax.axis_index("c")
    s = lax.axis_index("s")
    wid = c * 16 + s
    nva = wa // 16
    nvb = wb // 16
    sets = ((cs0, cd0, fa0, fb0, gsem0, ssem0),
            (cs1, cd1, fa1, fb1, gsem1, ssem1))

    zv = jnp.zeros((16,), jnp.float32)

    @pl.loop(0, ZB_ROWS)
    def _(r):
        for v in range(nva):
            fa0[r, pl.ds(v * 16, 16)] = zv
        for v in range(nvb):
            fb0[r, pl.ds(v * 16, 16)] = zv

    @pl.loop(s, N_BLOCKS, step=16)
    def _(k):
        pltpu.sync_copy(fa0, agga_sh.at[pl.ds(k * ZB_ROWS, ZB_ROWS)])
        pltpu.sync_copy(fb0, aggb_sh.at[pl.ds(k * ZB_ROWS, ZB_ROWS)])

    plsc.subcore_barrier()

    def issue(b, i):
        cs, cd, fa, fb, gsem, _ = sets[b]
        pltpu.async_copy(conf_h.at[sb.at[i]], cs, gsem)
        pltpu.async_copy(conf_h.at[db.at[i]], cd, gsem)
        pltpu.async_copy(ha_h.at[sb.at[i]], fa, gsem)
        pltpu.async_copy(hb_h.at[sb.at[i]], fb, gsem)

    def wait_gathers(b, i):
        cs, cd, fa, fb, gsem, _ = sets[b]
        pltpu.make_async_copy(conf_h.at[sb.at[i]], cs, gsem).wait()
        pltpu.make_async_copy(conf_h.at[db.at[i]], cd, gsem).wait()
        pltpu.make_async_copy(ha_h.at[sb.at[i]], fa, gsem).wait()
        pltpu.make_async_copy(hb_h.at[sb.at[i]], fb, gsem).wait()

    def scatter_async(b, i):
        cs, cd, fa, fb, _, ssem = sets[b]
        pltpu.async_copy(fa, agga_sh.at[db.at[i]], ssem, add=True)
        pltpu.async_copy(fb, aggb_sh.at[db.at[i]], ssem, add=True)

    def wait_scatter(b, i):
        cs, cd, fa, fb, _, ssem = sets[b]
        pltpu.make_async_copy(fa, agga_sh.at[db.at[i]], ssem).wait()
        pltpu.make_async_copy(fb, aggb_sh.at[db.at[i]], ssem).wait()

    dn = lax.GatherDimensionNumbers(
        offset_dims=(), collapsed_slice_dims=(0,), start_index_map=(0,))

    def compute(b):
        cs, cd, fa, fb, _, _ = sets[b]

        @pl.loop(0, C_CHUNK, unroll=2)
        def _(e):
            p = (cs[e, pl.ds(0, 16)] * cd[e, pl.ds(0, 16)]
                 + cs[e, pl.ds(16, 16)] * cd[e, pl.ds(16, 16)]
                 + cs[e, pl.ds(32, 16)] * cd[e, pl.ds(32, 16)])
            for stp in (8, 4, 2, 1):
                perm = (jnp.arange(16, dtype=jnp.int32) ^ stp)[:, None]
                p = p + lax.gather(p, perm, dimension_numbers=dn,
                                   slice_sizes=(1,),
                                   mode=lax.GatherScatterMode.PROMISE_IN_BOUNDS)
            for v in range(nva):
                fa[e, pl.ds(v * 16, 16)] = fa[e, pl.ds(v * 16, 16)] * p
            for v in range(nvb):
                fb[e, pl.ds(v * 16, 16)] = fb[e, pl.ds(v * 16, 16)] * p

    @pl.loop(0, N_CHUNKS // CPB)
    def _(q):
        row0 = wid * N_CHUNKS + q * CPB
        pltpu.sync_copy(src_h.at[pl.ds(row0, CPB)], sb)
        pltpu.sync_copy(dst_h.at[pl.ds(row0, CPB)], db)
        issue(0, 0)
        for i in range(CPB - 1):
            a = i % 2
            nb = 1 - a
            wait_gathers(a, i)
            if i >= 1:
                wait_scatter(nb, i - 1)
            issue(nb, i + 1)
            compute(a)
            scatter_async(a, i)
        a = (CPB - 1) % 2
        wait_gathers(a, CPB - 1)
        compute(a)
        scatter_async(a, CPB - 1)
        wait_scatter(1 - a, CPB - 2)
        wait_scatter(a, CPB - 1)

    plsc.subcore_barrier()

    @pl.loop(s, N_BLOCKS, step=16)
    def _(k):
        r = k * ZB_ROWS
        pltpu.sync_copy(agga_sh.at[pl.ds(r, ZB_ROWS)], fa0)
        pltpu.sync_copy(fa0, outa_h.at[c, pl.ds(r, ZB_ROWS)])
        pltpu.sync_copy(aggb_sh.at[pl.ds(r, ZB_ROWS)], fb0)
        pltpu.sync_copy(fb0, outb_h.at[c, pl.ds(r, ZB_ROWS)])


def _sc_agg(conf48, h2a, h2b, src, dst, wa, wb):
    mesh = plsc.VectorSubcoreMesh(core_axis_name="c", subcore_axis_name="s")
    kern = functools.partial(
        pl.kernel,
        out_type=[jax.ShapeDtypeStruct((2, N, wa), jnp.float32),
                  jax.ShapeDtypeStruct((2, N, wb), jnp.float32)],
        mesh=mesh,
        compiler_params=pltpu.CompilerParams(use_tc_tiling_on_sc=False),
        scratch_types=(
            [pltpu.VMEM((C_CHUNK, CONF_W), jnp.float32),
             pltpu.VMEM((C_CHUNK, CONF_W), jnp.float32),
             pltpu.VMEM((C_CHUNK, wa), jnp.float32),
             pltpu.VMEM((C_CHUNK, wb), jnp.float32)] * 2
            + [pltpu.VMEM((CPB, C_CHUNK), jnp.int32),
               pltpu.VMEM((CPB, C_CHUNK), jnp.int32)]
            + [pltpu.VMEM_SHARED((N, wa), jnp.float32),
               pltpu.VMEM_SHARED((N, wb), jnp.float32)]
            + [pltpu.SemaphoreType.DMA] * 4
        ),
    )(functools.partial(_sc_body, wa=wa, wb=wb))
    return kern(conf48, h2a, h2b, src, dst)


def kernel(x, edge_index, W0, b0, W1, b1, W2, b2, L0w, L0b, L1w, L1b, L2w,
           L2b, g0, be0, g1, be1):
    src = edge_index[0].reshape(E // C_CHUNK, C_CHUNK)
    dst = edge_index[1].reshape(E // C_CHUNK, C_CHUNK)

    conf0, ha0, hb0 = _tc_prep(x, L0w, L0b.reshape(1, -1), W0,
                               b0.reshape(1, -1), WA_HID, WB_HID)
    pa0, pb0 = _sc_agg(conf0, ha0, hb0, src, dst, WA_HID, WB_HID)
    conf1, ha1, hb1 = _tc_mid(pa0, pb0, ha0, hb0, g0.reshape(1, -1),
                              be0.reshape(1, -1), L1w, L1b.reshape(1, -1),
                              W1, b1.reshape(1, -1), WA_HID, WB_HID)
    pa1, pb1 = _sc_agg(conf1, ha1, hb1, src, dst, WA_HID, WB_HID)
    conf2, ha2, hb2 = _tc_mid(pa1, pb1, ha1, hb1, g1.reshape(1, -1),
                              be1.reshape(1, -1), L2w, L2b.reshape(1, -1),
                              W2, b2.reshape(1, -1), WA_OUT, WB_OUT)
    pa2, pb2 = _sc_agg(conf2, ha2, hb2, src, dst, WA_OUT, WB_OUT)
    return _tc_final(pa2, pb2, ha2, hb2)


# speedup vs baseline: 1.6159x; 1.1143x over previous
import functools

import jax
import jax.numpy as jnp
import numpy as np
from jax import lax
from jax.experimental import pallas as pl
from jax.experimental.pallas import tpu as pltpu
from jax.experimental.pallas import tpu_sc as plsc

N = 10000
E = 320000
D_IN = 128
D_HID = 128
D_OUT = 40
EPS = 1e-5
CONF_W = 48
C_CHUNK = 80
N_SUBCORES = 32
E_PER_W = E // N_SUBCORES
N_CHUNKS = E_PER_W // C_CHUNK
CPB = 5
ZB_ROWS = 80
N_BLOCKS = N // ZB_ROWS


def _dense_stage(hb, lw, lb, w, b, dp):
    logits = jnp.dot(hb, lw, preferred_element_type=jnp.float32) + lb
    m = jnp.max(logits, axis=1, keepdims=True)
    e = jnp.exp(logits - m)
    conf = e / jnp.sum(e, axis=1, keepdims=True)
    r = hb.shape[0]
    conf48 = jnp.concatenate(
        [conf, jnp.zeros((r, CONF_W - conf.shape[1]), jnp.float32)], axis=1)
    h2 = jnp.dot(hb, w, preferred_element_type=jnp.float32) + b
    do = h2.shape[1]
    pad = dp - do - 1
    h2p = jnp.concatenate(
        [h2, jnp.ones((r, 1), jnp.float32),
         jnp.zeros((r, pad), jnp.float32)], axis=1)
    return conf48, h2p


def _tc_prep_body(h_ref, lw_ref, lb_ref, w_ref, b_ref, conf_ref, h2p_ref, *, dp):
    conf48, h2p = _dense_stage(h_ref[...], lw_ref[...], lb_ref[...],
                               w_ref[...], b_ref[...], dp)
    conf_ref[...] = conf48
    h2p_ref[...] = h2p


def _tc_prep(h, lw, lb, w, b, dp, grid_r=1000):
    g = h.shape[0] // grid_r
    return pl.pallas_call(
        functools.partial(_tc_prep_body, dp=dp),
        grid=(g,),
        in_specs=[
            pl.BlockSpec((grid_r, h.shape[1]), lambda i: (i, 0)),
            pl.BlockSpec(lw.shape, lambda i: (0, 0)),
            pl.BlockSpec(lb.shape, lambda i: (0, 0)),
            pl.BlockSpec(w.shape, lambda i: (0, 0)),
            pl.BlockSpec(b.shape, lambda i: (0, 0)),
        ],
        out_specs=[
            pl.BlockSpec((grid_r, CONF_W), lambda i: (i, 0)),
            pl.BlockSpec((grid_r, dp), lambda i: (i, 0)),
        ],
        out_shape=[
            jax.ShapeDtypeStruct((h.shape[0], CONF_W), jnp.float32),
            jax.ShapeDtypeStruct((h.shape[0], dp), jnp.float32),
        ],
    )(h, lw, lb, w, b)


def _combine(part_ref, h2p_ref, do):
    tot = part_ref[0] + part_ref[1]
    h2 = h2p_ref[...][:, 0:do]
    agg = tot[:, 0:do]
    deg = tot[:, do:do + 1]
    return (h2 + agg) / (deg + 1.0)


def _tc_mid_body(part_ref, h2p_ref, g_ref, be_ref, lw_ref, lb_ref, w_ref,
                 b_ref, conf_ref, h2pn_ref, *, dp_next):
    hn = _combine(part_ref, h2p_ref, D_HID)
    hn = hn * (g_ref[...] * np.float32(1.0 / np.sqrt(1.0 + EPS))) + be_ref[...]
    hn = jnp.maximum(hn, 0.0)
    conf48, h2p = _dense_stage(hn, lw_ref[...], lb_ref[...], w_ref[...],
                               b_ref[...], dp_next)
    conf_ref[...] = conf48
    h2pn_ref[...] = h2p


def _tc_mid(part, h2p, g2d, be2d, lw, lb, w, b, dp_next, grid_r=1000):
    gr = N // grid_r
    dp = h2p.shape[1]
    return pl.pallas_call(
        functools.partial(_tc_mid_body, dp_next=dp_next),
        grid=(gr,),
        in_specs=[
            pl.BlockSpec((2, grid_r, dp), lambda i: (0, i, 0)),
            pl.BlockSpec((grid_r, dp), lambda i: (i, 0)),
            pl.BlockSpec(g2d.shape, lambda i: (0, 0)),
            pl.BlockSpec(be2d.shape, lambda i: (0, 0)),
            pl.BlockSpec(lw.shape, lambda i: (0, 0)),
            pl.BlockSpec(lb.shape, lambda i: (0, 0)),
            pl.BlockSpec(w.shape, lambda i: (0, 0)),
            pl.BlockSpec(b.shape, lambda i: (0, 0)),
        ],
        out_specs=[
            pl.BlockSpec((grid_r, CONF_W), lambda i: (i, 0)),
            pl.BlockSpec((grid_r, dp_next), lambda i: (i, 0)),
        ],
        out_shape=[
            jax.ShapeDtypeStruct((N, CONF_W), jnp.float32),
            jax.ShapeDtypeStruct((N, dp_next), jnp.float32),
        ],
    )(part, h2p, g2d, be2d, lw, lb, w, b)


def _tc_final_body(part_ref, h2p_ref, out_ref):
    out_ref[...] = _combine(part_ref, h2p_ref, D_OUT)


def _tc_final(part, h2p, grid_r=1000):
    gr = N // grid_r
    dp = h2p.shape[1]
    return pl.pallas_call(
        _tc_final_body,
        grid=(gr,),
        in_specs=[
            pl.BlockSpec((2, grid_r, dp), lambda i: (0, i, 0)),
            pl.BlockSpec((grid_r, dp), lambda i: (i, 0)),
        ],
        out_specs=pl.BlockSpec((grid_r, D_OUT), lambda i: (i, 0)),
        out_shape=jax.ShapeDtypeStruct((N, D_OUT), jnp.float32),
    )(part, h2p)


def _sc_body(conf_h, h2_h, src_h, dst_h, out_h,
             cs0, cd0, rows0, cs1, cd1, rows1, sb, db,
             agg_sh, gsem0, gsem1, ssem0, ssem1, *, dp):
    c = lax.axis_index("c")
    s = lax.axis_index("s")
    wid = c * 16 + s
    nv = dp // 16
    sets = ((cs0, cd0, rows0, gsem0, ssem0),
            (cs1, cd1, rows1, gsem1, ssem1))

    zv = jnp.zeros((16,), jnp.float32)

    @pl.loop(0, ZB_ROWS)
    def _(r):
        for v in range(nv):
            rows0[r, pl.ds(v * 16, 16)] = zv

    @pl.loop(s, N_BLOCKS, step=16)
    def _(k):
        pltpu.sync_copy(rows0, agg_sh.at[pl.ds(k * ZB_ROWS, ZB_ROWS)])

    plsc.subcore_barrier()

    def issue(b, i):
        cs, cd, rows, gsem, _ = sets[b]
        pltpu.async_copy(conf_h.at[sb.at[i]], cs, gsem)
        pltpu.async_copy(conf_h.at[db.at[i]], cd, gsem)
        pltpu.async_copy(h2_h.at[sb.at[i]], rows, gsem)

    def wait_gathers(b, i):
        cs, cd, rows, gsem, _ = sets[b]
        pltpu.make_async_copy(conf_h.at[sb.at[i]], cs, gsem).wait()
        pltpu.make_async_copy(conf_h.at[db.at[i]], cd, gsem).wait()
        pltpu.make_async_copy(h2_h.at[sb.at[i]], rows, gsem).wait()

    def scatter_async(b, i):
        cs, cd, rows, _, ssem = sets[b]
        pltpu.async_copy(rows, agg_sh.at[db.at[i]], ssem, add=True)

    def wait_scatter(b, i):
        cs, cd, rows, _, ssem = sets[b]
        pltpu.make_async_copy(rows, agg_sh.at[db.at[i]], ssem).wait()

    dn = lax.GatherDimensionNumbers(
        offset_dims=(), collapsed_slice_dims=(0,), start_index_map=(0,))

    def compute(b):
        cs, cd, rows, _, _ = sets[b]

        @pl.loop(0, C_CHUNK, unroll=4)
        def _(e):
            p = (cs[e, pl.ds(0, 16)] * cd[e, pl.ds(0, 16)]
                 + cs[e, pl.ds(16, 16)] * cd[e, pl.ds(16, 16)]
                 + cs[e, pl.ds(32, 16)] * cd[e, pl.ds(32, 16)])
            for stp in (8, 4, 2, 1):
                perm = (jnp.arange(16, dtype=jnp.int32) ^ stp)[:, None]
                p = p + lax.gather(p, perm, dimension_numbers=dn,
                                   slice_sizes=(1,),
                                   mode=lax.GatherScatterMode.PROMISE_IN_BOUNDS)
            for v in range(nv):
                rows[e, pl.ds(v * 16, 16)] = rows[e, pl.ds(v * 16, 16)] * p

    @pl.loop(0, N_CHUNKS // CPB)
    def _(q):
        row0 = wid * N_CHUNKS + q * CPB
        pltpu.sync_copy(src_h.at[pl.ds(row0, CPB)], sb)
        pltpu.sync_copy(dst_h.at[pl.ds(row0, CPB)], db)
        issue(0, 0)
        for i in range(CPB - 1):
            a = i % 2
            nb = 1 - a
            wait_gathers(a, i)
            if i >= 1:
                wait_scatter(nb, i - 1)
            issue(nb, i + 1)
            compute(a)
            scatter_async(a, i)
        a = (CPB - 1) % 2
        wait_gathers(a, CPB - 1)
        compute(a)
        scatter_async(a, CPB - 1)
        wait_scatter(1 - a, CPB - 2)
        wait_scatter(a, CPB - 1)

    plsc.subcore_barrier()

    @pl.loop(s, N_BLOCKS, step=16)
    def _(k):
        r = k * ZB_ROWS
        pltpu.sync_copy(agg_sh.at[pl.ds(r, ZB_ROWS)], rows0)
        pltpu.sync_copy(rows0, out_h.at[c, pl.ds(r, ZB_ROWS)])


def _sc_agg(conf48, h2p, src, dst, dp):
    mesh = plsc.VectorSubcoreMesh(core_axis_name="c", subcore_axis_name="s")
    kern = functools.partial(
        pl.kernel,
        out_type=jax.ShapeDtypeStruct((2, N, dp), jnp.float32),
        mesh=mesh,
        compiler_params=pltpu.CompilerParams(use_tc_tiling_on_sc=False),
        scratch_types=(
            [pltpu.VMEM((C_CHUNK, CONF_W), jnp.float32),
             pltpu.VMEM((C_CHUNK, CONF_W), jnp.float32),
             pltpu.VMEM((C_CHUNK, dp), jnp.float32)] * 2
            + [pltpu.VMEM((CPB, C_CHUNK), jnp.int32),
               pltpu.VMEM((CPB, C_CHUNK), jnp.int32)]
            + [pltpu.VMEM_SHARED((N, dp), jnp.float32)]
            + [pltpu.SemaphoreType.DMA] * 4
        ),
    )(functools.partial(_sc_body, dp=dp))
    return kern(conf48, h2p, src, dst)


def kernel(x, edge_index, W0, b0, W1, b1, W2, b2, L0w, L0b, L1w, L1b, L2w,
           L2b, g0, be0, g1, be1):
    src = edge_index[0].reshape(E // C_CHUNK, C_CHUNK)
    dst = edge_index[1].reshape(E // C_CHUNK, C_CHUNK)
    dp = D_HID + 16
    dp2 = D_OUT + 8

    conf0, h2p0 = _tc_prep(x, L0w, L0b.reshape(1, -1), W0, b0.reshape(1, -1), dp)
    part0 = _sc_agg(conf0, h2p0, src, dst, dp)
    conf1, h2p1 = _tc_mid(part0, h2p0, g0.reshape(1, -1), be0.reshape(1, -1),
                          L1w, L1b.reshape(1, -1), W1, b1.reshape(1, -1), dp)
    part1 = _sc_agg(conf1, h2p1, src, dst, dp)
    conf2, h2p2 = _tc_mid(part1, h2p1, g1.reshape(1, -1), be1.reshape(1, -1),
                          L2w, L2b.reshape(1, -1), W2, b2.reshape(1, -1), dp2)
    part2 = _sc_agg(conf2, h2p2, src, dst, dp2)
    return _tc_final(part2, h2p2)


# speedup vs baseline: 1.6435x; 1.0171x over previous
import functools

import jax
import jax.numpy as jnp
import numpy as np
from jax import lax
from jax.experimental import pallas as pl
from jax.experimental.pallas import tpu as pltpu
from jax.experimental.pallas import tpu_sc as plsc

N = 10000
E = 320000
D_IN = 128
D_HID = 128
D_OUT = 40
EPS = 1e-5
CONF_W = 48
C_CHUNK = 80
N_SUBCORES = 32
E_PER_W = E // N_SUBCORES
N_CHUNKS = E_PER_W // C_CHUNK
CPB = 5
ZB_ROWS = 80
N_BLOCKS = N // ZB_ROWS


def _dense_stage(hb, lw, lb, w, b, dp):
    logits = jnp.dot(hb, lw, preferred_element_type=jnp.float32) + lb
    m = jnp.max(logits, axis=1, keepdims=True)
    e = jnp.exp(logits - m)
    conf = e / jnp.sum(e, axis=1, keepdims=True)
    r = hb.shape[0]
    conf48 = jnp.concatenate(
        [conf, jnp.zeros((r, CONF_W - conf.shape[1]), jnp.float32)], axis=1)
    h2 = jnp.dot(hb, w, preferred_element_type=jnp.float32) + b
    do = h2.shape[1]
    pad = dp - do - 1
    h2p = jnp.concatenate(
        [h2, jnp.ones((r, 1), jnp.float32),
         jnp.zeros((r, pad), jnp.float32)], axis=1)
    return conf48, h2p


def _tc_prep_body(h_ref, lw_ref, lb_ref, w_ref, b_ref, conf_ref, h2p_ref, *, dp):
    conf48, h2p = _dense_stage(h_ref[...], lw_ref[...], lb_ref[...],
                               w_ref[...], b_ref[...], dp)
    conf_ref[...] = conf48
    h2p_ref[...] = h2p


def _tc_prep(h, lw, lb, w, b, dp, grid_r=2000):
    g = h.shape[0] // grid_r
    return pl.pallas_call(
        functools.partial(_tc_prep_body, dp=dp),
        grid=(g,),
        in_specs=[
            pl.BlockSpec((grid_r, h.shape[1]), lambda i: (i, 0)),
            pl.BlockSpec(lw.shape, lambda i: (0, 0)),
            pl.BlockSpec(lb.shape, lambda i: (0, 0)),
            pl.BlockSpec(w.shape, lambda i: (0, 0)),
            pl.BlockSpec(b.shape, lambda i: (0, 0)),
        ],
        out_specs=[
            pl.BlockSpec((grid_r, CONF_W), lambda i: (i, 0)),
            pl.BlockSpec((grid_r, dp), lambda i: (i, 0)),
        ],
        out_shape=[
            jax.ShapeDtypeStruct((h.shape[0], CONF_W), jnp.float32),
            jax.ShapeDtypeStruct((h.shape[0], dp), jnp.float32),
        ],
    )(h, lw, lb, w, b)


def _combine(part_ref, h2p_ref, do):
    tot = part_ref[0] + part_ref[1]
    h2 = h2p_ref[...][:, 0:do]
    agg = tot[:, 0:do]
    deg = tot[:, do:do + 1]
    return (h2 + agg) / (deg + 1.0)


def _tc_mid_body(part_ref, h2p_ref, g_ref, be_ref, lw_ref, lb_ref, w_ref,
                 b_ref, conf_ref, h2pn_ref, *, dp_next):
    hn = _combine(part_ref, h2p_ref, D_HID)
    hn = hn * (g_ref[...] * np.float32(1.0 / np.sqrt(1.0 + EPS))) + be_ref[...]
    hn = jnp.maximum(hn, 0.0)
    conf48, h2p = _dense_stage(hn, lw_ref[...], lb_ref[...], w_ref[...],
                               b_ref[...], dp_next)
    conf_ref[...] = conf48
    h2pn_ref[...] = h2p


def _tc_mid(part, h2p, g2d, be2d, lw, lb, w, b, dp_next, grid_r=2000):
    gr = N // grid_r
    dp = h2p.shape[1]
    return pl.pallas_call(
        functools.partial(_tc_mid_body, dp_next=dp_next),
        grid=(gr,),
        in_specs=[
            pl.BlockSpec((2, grid_r, dp), lambda i: (0, i, 0)),
            pl.BlockSpec((grid_r, dp), lambda i: (i, 0)),
            pl.BlockSpec(g2d.shape, lambda i: (0, 0)),
            pl.BlockSpec(be2d.shape, lambda i: (0, 0)),
            pl.BlockSpec(lw.shape, lambda i: (0, 0)),
            pl.BlockSpec(lb.shape, lambda i: (0, 0)),
            pl.BlockSpec(w.shape, lambda i: (0, 0)),
            pl.BlockSpec(b.shape, lambda i: (0, 0)),
        ],
        out_specs=[
            pl.BlockSpec((grid_r, CONF_W), lambda i: (i, 0)),
            pl.BlockSpec((grid_r, dp_next), lambda i: (i, 0)),
        ],
        out_shape=[
            jax.ShapeDtypeStruct((N, CONF_W), jnp.float32),
            jax.ShapeDtypeStruct((N, dp_next), jnp.float32),
        ],
    )(part, h2p, g2d, be2d, lw, lb, w, b)


def _tc_final_body(part_ref, h2p_ref, out_ref):
    out_ref[...] = _combine(part_ref, h2p_ref, D_OUT)


def _tc_final(part, h2p, grid_r=2000):
    gr = N // grid_r
    dp = h2p.shape[1]
    return pl.pallas_call(
        _tc_final_body,
        grid=(gr,),
        in_specs=[
            pl.BlockSpec((2, grid_r, dp), lambda i: (0, i, 0)),
            pl.BlockSpec((grid_r, dp), lambda i: (i, 0)),
        ],
        out_specs=pl.BlockSpec((grid_r, D_OUT), lambda i: (i, 0)),
        out_shape=jax.ShapeDtypeStruct((N, D_OUT), jnp.float32),
    )(part, h2p)


def _sc_body(conf_h, h2_h, src_h, dst_h, out_h,
             cs0, cd0, rows0, cs1, cd1, rows1, sb, db,
             agg_sh, gsem0, gsem1, ssem0, ssem1, *, dp):
    c = lax.axis_index("c")
    s = lax.axis_index("s")
    wid = c * 16 + s
    nv = dp // 16
    sets = ((cs0, cd0, rows0, gsem0, ssem0),
            (cs1, cd1, rows1, gsem1, ssem1))

    zv = jnp.zeros((16,), jnp.float32)

    @pl.loop(0, ZB_ROWS)
    def _(r):
        for v in range(nv):
            rows0[r, pl.ds(v * 16, 16)] = zv

    @pl.loop(s, N_BLOCKS, step=16)
    def _(k):
        pltpu.sync_copy(rows0, agg_sh.at[pl.ds(k * ZB_ROWS, ZB_ROWS)])

    plsc.subcore_barrier()

    def issue(b, i):
        cs, cd, rows, gsem, _ = sets[b]
        pltpu.async_copy(conf_h.at[sb.at[i]], cs, gsem)
        pltpu.async_copy(conf_h.at[db.at[i]], cd, gsem)
        pltpu.async_copy(h2_h.at[sb.at[i]], rows, gsem)

    def wait_gathers(b, i):
        cs, cd, rows, gsem, _ = sets[b]
        pltpu.make_async_copy(conf_h.at[sb.at[i]], cs, gsem).wait()
        pltpu.make_async_copy(conf_h.at[db.at[i]], cd, gsem).wait()
        pltpu.make_async_copy(h2_h.at[sb.at[i]], rows, gsem).wait()

    def scatter_async(b, i):
        cs, cd, rows, _, ssem = sets[b]
        pltpu.async_copy(rows, agg_sh.at[db.at[i]], ssem, add=True)

    def wait_scatter(b, i):
        cs, cd, rows, _, ssem = sets[b]
        pltpu.make_async_copy(rows, agg_sh.at[db.at[i]], ssem).wait()

    dn = lax.GatherDimensionNumbers(
        offset_dims=(), collapsed_slice_dims=(0,), start_index_map=(0,))

    def compute(b):
        cs, cd, rows, _, _ = sets[b]

        @pl.loop(0, C_CHUNK, unroll=2)
        def _(e):
            p = (cs[e, pl.ds(0, 16)] * cd[e, pl.ds(0, 16)]
                 + cs[e, pl.ds(16, 16)] * cd[e, pl.ds(16, 16)]
                 + cs[e, pl.ds(32, 16)] * cd[e, pl.ds(32, 16)])
            for stp in (8, 4, 2, 1):
                perm = (jnp.arange(16, dtype=jnp.int32) ^ stp)[:, None]
                p = p + lax.gather(p, perm, dimension_numbers=dn,
                                   slice_sizes=(1,),
                                   mode=lax.GatherScatterMode.PROMISE_IN_BOUNDS)
            for v in range(nv):
                rows[e, pl.ds(v * 16, 16)] = rows[e, pl.ds(v * 16, 16)] * p

    @pl.loop(0, N_CHUNKS // CPB)
    def _(q):
        row0 = wid * N_CHUNKS + q * CPB
        pltpu.sync_copy(src_h.at[pl.ds(row0, CPB)], sb)
        pltpu.sync_copy(dst_h.at[pl.ds(row0, CPB)], db)
        issue(0, 0)
        for i in range(CPB - 1):
            a = i % 2
            nb = 1 - a
            wait_gathers(a, i)
            if i >= 1:
                wait_scatter(nb, i - 1)
            issue(nb, i + 1)
            compute(a)
            scatter_async(a, i)
        a = (CPB - 1) % 2
        wait_gathers(a, CPB - 1)
        compute(a)
        scatter_async(a, CPB - 1)
        wait_scatter(1 - a, CPB - 2)
        wait_scatter(a, CPB - 1)

    plsc.subcore_barrier()

    @pl.loop(s, N_BLOCKS, step=16)
    def _(k):
        r = k * ZB_ROWS
        pltpu.sync_copy(agg_sh.at[pl.ds(r, ZB_ROWS)], rows0)
        pltpu.sync_copy(rows0, out_h.at[c, pl.ds(r, ZB_ROWS)])


def _sc_agg(conf48, h2p, src, dst, dp):
    mesh = plsc.VectorSubcoreMesh(core_axis_name="c", subcore_axis_name="s")
    kern = functools.partial(
        pl.kernel,
        out_type=jax.ShapeDtypeStruct((2, N, dp), jnp.float32),
        mesh=mesh,
        compiler_params=pltpu.CompilerParams(use_tc_tiling_on_sc=False),
        scratch_types=(
            [pltpu.VMEM((C_CHUNK, CONF_W), jnp.float32),
             pltpu.VMEM((C_CHUNK, CONF_W), jnp.float32),
             pltpu.VMEM((C_CHUNK, dp), jnp.float32)] * 2
            + [pltpu.VMEM((CPB, C_CHUNK), jnp.int32),
               pltpu.VMEM((CPB, C_CHUNK), jnp.int32)]
            + [pltpu.VMEM_SHARED((N, dp), jnp.float32)]
            + [pltpu.SemaphoreType.DMA] * 4
        ),
    )(functools.partial(_sc_body, dp=dp))
    return kern(conf48, h2p, src, dst)


def kernel(x, edge_index, W0, b0, W1, b1, W2, b2, L0w, L0b, L1w, L1b, L2w,
           L2b, g0, be0, g1, be1):
    src = edge_index[0].reshape(E // C_CHUNK, C_CHUNK)
    dst = edge_index[1].reshape(E // C_CHUNK, C_CHUNK)
    dp = D_HID + 16
    dp2 = D_OUT + 8

    conf0, h2p0 = _tc_prep(x, L0w, L0b.reshape(1, -1), W0, b0.reshape(1, -1), dp)
    part0 = _sc_agg(conf0, h2p0, src, dst, dp)
    conf1, h2p1 = _tc_mid(part0, h2p0, g0.reshape(1, -1), be0.reshape(1, -1),
                          L1w, L1b.reshape(1, -1), W1, b1.reshape(1, -1), dp)
    part1 = _sc_agg(conf1, h2p1, src, dst, dp)
    conf2, h2p2 = _tc_mid(part1, h2p1, g1.reshape(1, -1), be1.reshape(1, -1),
                          L2w, L2b.reshape(1, -1), W2, b2.reshape(1, -1), dp2)
    part2 = _sc_agg(conf2, h2p2, src, dst, dp2)
    return _tc_final(part2, h2p2)


# speedup vs baseline: 1.6436x; 1.0001x over previous
import functools

import jax
import jax.numpy as jnp
import numpy as np
from jax import lax
from jax.experimental import pallas as pl
from jax.experimental.pallas import tpu as pltpu
from jax.experimental.pallas import tpu_sc as plsc

N = 10000
E = 320000
D_IN = 128
D_HID = 128
D_OUT = 40
EPS = 1e-5
CONF_W = 48
C_CHUNK = 80
N_SUBCORES = 32
E_PER_W = E // N_SUBCORES
N_CHUNKS = E_PER_W // C_CHUNK
CPB = 5
ZB_ROWS = 80
N_BLOCKS = N // ZB_ROWS


def _dense_stage(hb, lw, lb, w, b, dp):
    logits = jnp.dot(hb, lw, preferred_element_type=jnp.float32) + lb
    m = jnp.max(logits, axis=1, keepdims=True)
    e = jnp.exp(logits - m)
    conf = e / jnp.sum(e, axis=1, keepdims=True)
    r = hb.shape[0]
    conf48 = jnp.concatenate(
        [conf, jnp.zeros((r, CONF_W - conf.shape[1]), jnp.float32)], axis=1)
    h2 = jnp.dot(hb, w, preferred_element_type=jnp.float32) + b
    do = h2.shape[1]
    pad = dp - do - 1
    h2p = jnp.concatenate(
        [h2, jnp.ones((r, 1), jnp.float32),
         jnp.zeros((r, pad), jnp.float32)], axis=1)
    return conf48, h2p


def _tc_prep_body(h_ref, lw_ref, lb_ref, w_ref, b_ref, conf_ref, h2p_ref, *, dp):
    conf48, h2p = _dense_stage(h_ref[...], lw_ref[...], lb_ref[...],
                               w_ref[...], b_ref[...], dp)
    conf_ref[...] = conf48
    h2p_ref[...] = h2p


def _tc_prep(h, lw, lb, w, b, dp, grid_r=2000):
    g = h.shape[0] // grid_r
    return pl.pallas_call(
        functools.partial(_tc_prep_body, dp=dp),
        grid=(g,),
        in_specs=[
            pl.BlockSpec((grid_r, h.shape[1]), lambda i: (i, 0)),
            pl.BlockSpec(lw.shape, lambda i: (0, 0)),
            pl.BlockSpec(lb.shape, lambda i: (0, 0)),
            pl.BlockSpec(w.shape, lambda i: (0, 0)),
            pl.BlockSpec(b.shape, lambda i: (0, 0)),
        ],
        out_specs=[
            pl.BlockSpec((grid_r, CONF_W), lambda i: (i, 0)),
            pl.BlockSpec((grid_r, dp), lambda i: (i, 0)),
        ],
        out_shape=[
            jax.ShapeDtypeStruct((h.shape[0], CONF_W), jnp.float32),
            jax.ShapeDtypeStruct((h.shape[0], dp), jnp.float32),
        ],
    )(h, lw, lb, w, b)


def _combine(part_ref, h2p_ref, do):
    tot = part_ref[0] + part_ref[1]
    h2 = h2p_ref[...][:, 0:do]
    agg = tot[:, 0:do]
    deg = tot[:, do:do + 1]
    return (h2 + agg) / (deg + 1.0)


def _tc_mid_body(part_ref, h2p_ref, g_ref, be_ref, lw_ref, lb_ref, w_ref,
                 b_ref, conf_ref, h2pn_ref, *, dp_next):
    hn = _combine(part_ref, h2p_ref, D_HID)
    hn = hn * (g_ref[...] * np.float32(1.0 / np.sqrt(1.0 + EPS))) + be_ref[...]
    hn = jnp.maximum(hn, 0.0)
    conf48, h2p = _dense_stage(hn, lw_ref[...], lb_ref[...], w_ref[...],
                               b_ref[...], dp_next)
    conf_ref[...] = conf48
    h2pn_ref[...] = h2p


def _tc_mid(part, h2p, g2d, be2d, lw, lb, w, b, dp_next, grid_r=2000):
    gr = N // grid_r
    dp = h2p.shape[1]
    return pl.pallas_call(
        functools.partial(_tc_mid_body, dp_next=dp_next),
        grid=(gr,),
        in_specs=[
            pl.BlockSpec((2, grid_r, dp), lambda i: (0, i, 0)),
            pl.BlockSpec((grid_r, dp), lambda i: (i, 0)),
            pl.BlockSpec(g2d.shape, lambda i: (0, 0)),
            pl.BlockSpec(be2d.shape, lambda i: (0, 0)),
            pl.BlockSpec(lw.shape, lambda i: (0, 0)),
            pl.BlockSpec(lb.shape, lambda i: (0, 0)),
            pl.BlockSpec(w.shape, lambda i: (0, 0)),
            pl.BlockSpec(b.shape, lambda i: (0, 0)),
        ],
        out_specs=[
            pl.BlockSpec((grid_r, CONF_W), lambda i: (i, 0)),
            pl.BlockSpec((grid_r, dp_next), lambda i: (i, 0)),
        ],
        out_shape=[
            jax.ShapeDtypeStruct((N, CONF_W), jnp.float32),
            jax.ShapeDtypeStruct((N, dp_next), jnp.float32),
        ],
    )(part, h2p, g2d, be2d, lw, lb, w, b)


def _tc_final_body(part_ref, h2p_ref, out_ref):
    out_ref[...] = _combine(part_ref, h2p_ref, D_OUT)


def _tc_final(part, h2p, grid_r=2000):
    gr = N // grid_r
    dp = h2p.shape[1]
    return pl.pallas_call(
        _tc_final_body,
        grid=(gr,),
        in_specs=[
            pl.BlockSpec((2, grid_r, dp), lambda i: (0, i, 0)),
            pl.BlockSpec((grid_r, dp), lambda i: (i, 0)),
        ],
        out_specs=pl.BlockSpec((grid_r, D_OUT), lambda i: (i, 0)),
        out_shape=jax.ShapeDtypeStruct((N, D_OUT), jnp.float32),
    )(part, h2p)


def _sc_body(conf_h, h2_h, src_h, dst_h, out_h,
             cs0, cd0, rows0, cs1, cd1, rows1, sb, db,
             agg_sh, gsem0, gsem1, ssem0, ssem1, *, dp):
    c = lax.axis_index("c")
    s = lax.axis_index("s")
    wid = c * 16 + s
    nv = dp // 16
    sets = ((cs0, cd0, rows0, gsem0, ssem0),
            (cs1, cd1, rows1, gsem1, ssem1))

    zv = jnp.zeros((16,), jnp.float32)

    @pl.loop(0, ZB_ROWS)
    def _(r):
        for v in range(nv):
            rows0[r, pl.ds(v * 16, 16)] = zv

    @pl.loop(s, N_BLOCKS, step=16)
    def _(k):
        pltpu.sync_copy(rows0, agg_sh.at[pl.ds(k * ZB_ROWS, ZB_ROWS)])

    plsc.subcore_barrier()

    def issue(b, i):
        cs, cd, rows, gsem, _ = sets[b]
        pltpu.async_copy(conf_h.at[sb.at[i]], cs, gsem)
        pltpu.async_copy(conf_h.at[db.at[i]], cd, gsem)
        pltpu.async_copy(h2_h.at[sb.at[i]], rows, gsem)

    def wait_gathers(b, i):
        cs, cd, rows, gsem, _ = sets[b]
        pltpu.make_async_copy(conf_h.at[sb.at[i]], cs, gsem).wait()
        pltpu.make_async_copy(conf_h.at[db.at[i]], cd, gsem).wait()
        pltpu.make_async_copy(h2_h.at[sb.at[i]], rows, gsem).wait()

    def scatter_async(b, i):
        cs, cd, rows, _, ssem = sets[b]
        pltpu.async_copy(rows, agg_sh.at[db.at[i]], ssem, add=True)

    def wait_scatter(b, i):
        cs, cd, rows, _, ssem = sets[b]
        pltpu.make_async_copy(rows, agg_sh.at[db.at[i]], ssem).wait()

    dn = lax.GatherDimensionNumbers(
        offset_dims=(), collapsed_slice_dims=(0,), start_index_map=(0,))

    def compute(b):
        cs, cd, rows, _, _ = sets[b]

        @pl.loop(0, C_CHUNK, unroll=2)
        def _(e):
            p = (cs[e, pl.ds(0, 16)] * cd[e, pl.ds(0, 16)]
                 + cs[e, pl.ds(16, 16)] * cd[e, pl.ds(16, 16)]
                 + cs[e, pl.ds(32, 16)] * cd[e, pl.ds(32, 16)])
            for stp in (8, 4, 2, 1):
                perm = (jnp.arange(16, dtype=jnp.int32) ^ stp)[:, None]
                p = p + lax.gather(p, perm, dimension_numbers=dn,
                                   slice_sizes=(1,),
                                   mode=lax.GatherScatterMode.PROMISE_IN_BOUNDS)
            for v in range(nv):
                rows[e, pl.ds(v * 16, 16)] = rows[e, pl.ds(v * 16, 16)] * p

    @pl.loop(0, N_CHUNKS // CPB)
    def _(q):
        row0 = wid * N_CHUNKS + q * CPB
        pltpu.sync_copy(src_h.at[pl.ds(row0, CPB)], sb)
        pltpu.sync_copy(dst_h.at[pl.ds(row0, CPB)], db)
        issue(0, 0)
        for i in range(CPB - 1):
            a = i % 2
            nb = 1 - a
            wait_gathers(a, i)
            if i >= 1:
                wait_scatter(nb, i - 1)
            issue(nb, i + 1)
            compute(a)
            scatter_async(a, i)
        a = (CPB - 1) % 2
        wait_gathers(a, CPB - 1)
        compute(a)
        scatter_async(a, CPB - 1)
        wait_scatter(1 - a, CPB - 2)
        wait_scatter(a, CPB - 1)

    plsc.subcore_barrier()

    @pl.loop(s, N_BLOCKS, step=16)
    def _(k):
        r = k * ZB_ROWS
        pltpu.sync_copy(agg_sh.at[pl.ds(r, ZB_ROWS)],
                        out_h.at[c, pl.ds(r, ZB_ROWS)])


def _sc_agg(conf48, h2p, src, dst, dp):
    mesh = plsc.VectorSubcoreMesh(core_axis_name="c", subcore_axis_name="s")
    kern = functools.partial(
        pl.kernel,
        out_type=jax.ShapeDtypeStruct((2, N, dp), jnp.float32),
        mesh=mesh,
        compiler_params=pltpu.CompilerParams(use_tc_tiling_on_sc=False),
        scratch_types=(
            [pltpu.VMEM((C_CHUNK, CONF_W), jnp.float32),
             pltpu.VMEM((C_CHUNK, CONF_W), jnp.float32),
             pltpu.VMEM((C_CHUNK, dp), jnp.float32)] * 2
            + [pltpu.VMEM((CPB, C_CHUNK), jnp.int32),
               pltpu.VMEM((CPB, C_CHUNK), jnp.int32)]
            + [pltpu.VMEM_SHARED((N, dp), jnp.float32)]
            + [pltpu.SemaphoreType.DMA] * 4
        ),
    )(functools.partial(_sc_body, dp=dp))
    return kern(conf48, h2p, src, dst)


def kernel(x, edge_index, W0, b0, W1, b1, W2, b2, L0w, L0b, L1w, L1b, L2w,
           L2b, g0, be0, g1, be1):
    src = edge_index[0].reshape(E // C_CHUNK, C_CHUNK)
    dst = edge_index[1].reshape(E // C_CHUNK, C_CHUNK)
    dp = D_HID + 16
    dp2 = D_OUT + 8

    conf0, h2p0 = _tc_prep(x, L0w, L0b.reshape(1, -1), W0, b0.reshape(1, -1), dp)
    part0 = _sc_agg(conf0, h2p0, src, dst, dp)
    conf1, h2p1 = _tc_mid(part0, h2p0, g0.reshape(1, -1), be0.reshape(1, -1),
                          L1w, L1b.reshape(1, -1), W1, b1.reshape(1, -1), dp)
    part1 = _sc_agg(conf1, h2p1, src, dst, dp)
    conf2, h2p2 = _tc_mid(part1, h2p1, g1.reshape(1, -1), be1.reshape(1, -1),
                          L2w, L2b.reshape(1, -1), W2, b2.reshape(1, -1), dp2)
    part2 = _sc_agg(conf2, h2p2, src, dst, dp2)
    return _tc_final(part2, h2p2)
